# baseline (speedup 1.0000x reference)
import jax
import jax.numpy as jnp
from jax import lax
from jax.experimental import pallas as pl
from jax.experimental.pallas import tpu as pltpu
from jax.experimental.pallas import tpu_sc as plsc

_NUM_CORES = 2
_NUM_SUBCORES = 16
_NUM_WORKERS = _NUM_CORES * _NUM_SUBCORES
_LANES = 16
_SLICES = 4

_COLS = 1024
_SC_ROWS = 256
_TC_BLOCK_ROWS = 128


def _sc_body(x_hbm, check_hbm, out_hbm, check_v, table_v, x_v, out_v, *sems):
    c = check_hbm.shape[0]
    m = out_hbm.shape[0]
    per_w = m // _NUM_WORKERS
    per_s = per_w // _SLICES
    wid = lax.axis_index("s") * _NUM_CORES + lax.axis_index("c")
    base = wid * per_w
    isems = sems[:_SLICES]
    osems = sems[_SLICES:2 * _SLICES]
    csem = sems[-1]

    in_d = [
        pltpu.async_copy(
            x_hbm.at[pl.ds(base + si * per_s, per_s)],
            x_v.at[pl.ds(si * per_s, per_s)],
            isems[si],
        )
        for si in range(_SLICES)
    ]

    check_d = pltpu.async_copy(check_hbm, check_v, csem)
    for j0 in range(c // _LANES):
        table_v[pl.ds(j0 * _LANES, _LANES)] = jnp.zeros((_LANES,), jnp.int32)
    check_d.wait()
    for j0 in reversed(range(c // _LANES)):
        vals = check_v[pl.ds(j0 * _LANES, _LANES)]
        js = lax.iota(jnp.int32, _LANES) + (j0 * _LANES)
        plsc.store_scatter(table_v, [vals], js)

    out_d = []
    for si in range(_SLICES):
        in_d[si].wait()

        @plsc.parallel_loop(si * per_s // _LANES,
                            (si + 1) * per_s // _LANES, unroll=8)
        def _(i):
            sl = pl.ds(i * _LANES, _LANES)
            out_v[sl] = plsc.load_gather(table_v, [x_v[sl]])

        out_d.append(
            pltpu.async_copy(
                out_v.at[pl.ds(si * per_s, per_s)],
                out_hbm.at[pl.ds(base + si * per_s, per_s)],
                osems[si],
            )
        )
    for d in out_d:
        d.wait()


def _sc_translate(x, check_tensor, m):
    per_w = m // _NUM_WORKERS
    mesh = plsc.VectorSubcoreMesh(
        core_axis_name="c",
        subcore_axis_name="s",
        num_cores=_NUM_CORES,
        num_subcores=_NUM_SUBCORES,
    )
    f = pl.kernel(
        _sc_body,
        out_type=jax.ShapeDtypeStruct((m,), jnp.int32),
        mesh=mesh,
        scratch_types=[
            pltpu.VMEM((check_tensor.shape[0],), jnp.int32),
            pltpu.VMEM((check_tensor.shape[0],), jnp.int32),
            pltpu.VMEM((per_w,), jnp.int32),
            pltpu.VMEM((per_w,), jnp.int32),
        ] + [pltpu.SemaphoreType.DMA] * (2 * _SLICES + 1),
        compiler_params=pltpu.CompilerParams(needs_layout_passes=False),
    )
    return f(x, check_tensor)


def _tc_block_body(check_ref, x_ref, o_ref):
    c = check_ref.shape[0]
    x = x_ref[...]
    acc = jnp.zeros_like(x)
    for j in reversed(range(c)):
        acc = jnp.where(x == check_ref[j], j, acc)
    o_ref[...] = acc


def _tc_translate(x2, check_tensor):
    rows = x2.shape[0]
    tc_rows = rows - _SC_ROWS
    grid = tc_rows // _TC_BLOCK_ROWS
    skip = _SC_ROWS // _TC_BLOCK_ROWS
    out2 = pl.pallas_call(
        _tc_block_body,
        grid=(grid,),
        in_specs=[
            pl.BlockSpec(memory_space=pltpu.SMEM),
            pl.BlockSpec((_TC_BLOCK_ROWS, _COLS), lambda i: (skip + i, 0)),
        ],
        out_specs=pl.BlockSpec((_TC_BLOCK_ROWS, _COLS), lambda i: (i, 0)),
        out_shape=jax.ShapeDtypeStruct((tc_rows, _COLS), jnp.int32),
    )(check_tensor, x2)
    return out2.reshape(tc_rows * _COLS)


def kernel(x, check_tensor):
    n = x.shape[0]
    m = _SC_ROWS * _COLS
    x2 = x.reshape(n // _COLS, _COLS)
    out_sc = _sc_translate(x, check_tensor, m)
    out_tc = _tc_translate(x2, check_tensor)
    return jnp.concatenate([out_sc, out_tc])


# speedup vs baseline: 1.6893x; 1.6893x over previous
import jax
import jax.numpy as jnp
from jax import lax
from jax.experimental import pallas as pl
from jax.experimental.pallas import tpu as pltpu
from jax.experimental.pallas import tpu_sc as plsc

_NUM_CORES = 2
_NUM_SUBCORES = 16
_NUM_WORKERS = _NUM_CORES * _NUM_SUBCORES
_LANES = 16
_SLICES = 4


def _body(x_hbm, check_hbm, out_hbm, check_v, table_v, x_v, out_v, *sems):
    n = x_hbm.shape[0]
    c = check_hbm.shape[0]
    per_w = n // _NUM_WORKERS
    per_s = per_w // _SLICES
    wid = lax.axis_index("s") * _NUM_CORES + lax.axis_index("c")
    base = wid * per_w
    isems = sems[:_SLICES]
    osems = sems[_SLICES:2 * _SLICES]
    csem = sems[-1]

    in_d = [
        pltpu.async_copy(
            x_hbm.at[pl.ds(base + si * per_s, per_s)],
            x_v.at[pl.ds(si * per_s, per_s)],
            isems[si],
        )
        for si in range(_SLICES)
    ]

    check_d = pltpu.async_copy(check_hbm, check_v, csem)
    for j0 in range(c // _LANES):
        table_v[pl.ds(j0 * _LANES, _LANES)] = jnp.zeros((_LANES,), jnp.int32)
    check_d.wait()
    for j0 in reversed(range(c // _LANES)):
        vals = check_v[pl.ds(j0 * _LANES, _LANES)]
        js = lax.iota(jnp.int32, _LANES) + (j0 * _LANES)
        plsc.store_scatter(table_v, [vals], js)

    out_d = []
    for si in range(_SLICES):
        in_d[si].wait()

        @plsc.parallel_loop(si * per_s // _LANES,
                            (si + 1) * per_s // _LANES, unroll=8)
        def _(i):
            sl = pl.ds(i * _LANES, _LANES)
            out_v[sl] = plsc.load_gather(table_v, [x_v[sl]])

        out_d.append(
            pltpu.async_copy(
                out_v.at[pl.ds(si * per_s, per_s)],
                out_hbm.at[pl.ds(base + si * per_s, per_s)],
                osems[si],
            )
        )
    for d in out_d:
        d.wait()


def kernel(x, check_tensor):
    n = x.shape[0]
    per_w = n // _NUM_WORKERS
    mesh = plsc.VectorSubcoreMesh(
        core_axis_name="c",
        subcore_axis_name="s",
        num_cores=_NUM_CORES,
        num_subcores=_NUM_SUBCORES,
    )
    f = pl.kernel(
        _body,
        out_type=jax.ShapeDtypeStruct((n,), jnp.int32),
        mesh=mesh,
        scratch_types=[
            pltpu.VMEM((check_tensor.shape[0],), jnp.int32),
            pltpu.VMEM((check_tensor.shape[0],), jnp.int32),
            pltpu.VMEM((per_w,), jnp.int32),
            pltpu.VMEM((per_w,), jnp.int32),
        ] + [pltpu.SemaphoreType.DMA] * (2 * _SLICES + 1),
        compiler_params=pltpu.CompilerParams(needs_layout_passes=False),
    )
    return f(x, check_tensor)


# speedup vs baseline: 1.6953x; 1.0035x over previous
import jax
import jax.numpy as jnp
from jax import lax
from jax.experimental import pallas as pl
from jax.experimental.pallas import tpu as pltpu
from jax.experimental.pallas import tpu_sc as plsc

_NUM_CORES = 2
_NUM_SUBCORES = 16
_NUM_WORKERS = _NUM_CORES * _NUM_SUBCORES
_LANES = 16
_SLICES = 2


def _body(x_hbm, check_hbm, out_hbm, check_v, table_v, x_v, out_v, *sems):
    n = x_hbm.shape[0]
    c = check_hbm.shape[0]
    per_w = n // _NUM_WORKERS
    per_s = per_w // _SLICES
    wid = lax.axis_index("s") * _NUM_CORES + lax.axis_index("c")
    base = wid * per_w
    isems = sems[:_SLICES]
    osems = sems[_SLICES:2 * _SLICES]
    csem = sems[-1]

    in_d = [
        pltpu.async_copy(
            x_hbm.at[pl.ds(base + si * per_s, per_s)],
            x_v.at[pl.ds(si * per_s, per_s)],
            isems[si],
        )
        for si in range(_SLICES)
    ]

    check_d = pltpu.async_copy(check_hbm, check_v, csem)
    for j0 in range(c // _LANES):
        table_v[pl.ds(j0 * _LANES, _LANES)] = jnp.zeros((_LANES,), jnp.int32)
    check_d.wait()
    for j0 in reversed(range(c // _LANES)):
        vals = check_v[pl.ds(j0 * _LANES, _LANES)]
        js = lax.iota(jnp.int32, _LANES) + (j0 * _LANES)
        plsc.store_scatter(table_v, [vals], js)

    out_d = []
    for si in range(_SLICES):
        in_d[si].wait()

        @plsc.parallel_loop(si * per_s // _LANES,
                            (si + 1) * per_s // _LANES, unroll=8)
        def _(i):
            sl = pl.ds(i * _LANES, _LANES)
            out_v[sl] = plsc.load_gather(table_v, [x_v[sl]])

        out_d.append(
            pltpu.async_copy(
                out_v.at[pl.ds(si * per_s, per_s)],
                out_hbm.at[pl.ds(base + si * per_s, per_s)],
                osems[si],
            )
        )
    for d in out_d:
        d.wait()


def kernel(x, check_tensor):
    n = x.shape[0]
    per_w = n // _NUM_WORKERS
    mesh = plsc.VectorSubcoreMesh(
        core_axis_name="c",
        subcore_axis_name="s",
        num_cores=_NUM_CORES,
        num_subcores=_NUM_SUBCORES,
    )
    f = pl.kernel(
        _body,
        out_type=jax.ShapeDtypeStruct((n,), jnp.int32),
        mesh=mesh,
        scratch_types=[
            pltpu.VMEM((check_tensor.shape[0],), jnp.int32),
            pltpu.VMEM((check_tensor.shape[0],), jnp.int32),
            pltpu.VMEM((per_w,), jnp.int32),
            pltpu.VMEM((per_w,), jnp.int32),
        ] + [pltpu.SemaphoreType.DMA] * (2 * _SLICES + 1),
        compiler_params=pltpu.CompilerParams(needs_layout_passes=False),
    )
    return f(x, check_tensor)
